# initial kernel scaffold (unmeasured)
import jax
import jax.numpy as jnp
from jax import lax
from jax.experimental import pallas as pl
from jax.experimental.pallas import tpu as pltpu

N_DEV = 4
TILE_N = 2048


def kernel(x, w_mat, scale_x, scale_w):
    m, _k = x.shape
    _kw, n = w_mat.shape
    m_out = m // N_DEV
    n_tiles = n // TILE_N
    n_hops = N_DEV - 1

    def body(x_ref, w_ref, sx_ref, sw_ref, out_ref,
             send_ref, sscale_ref, recv_ref, rscale_ref, stage_ref,
             send_sem, sscale_sem, recv_sems, rscale_sems, copy_sem):
        my = lax.axis_index("i")
        right = lax.rem(my + 1, N_DEV)
        left = lax.rem(my + N_DEV - 1, N_DEV)

        barrier = pltpu.get_barrier_semaphore()
        for nbr in (left, right):
            pl.semaphore_signal(barrier, inc=1, device_id=(nbr,),
                                device_id_type=pl.DeviceIdType.MESH)
        pl.semaphore_wait(barrier, 2)

        def contrib_tile(j, nt):
            xj = x_ref[pl.ds(j * m_out, m_out), :].astype(jnp.bfloat16)
            wt = w_ref[:, nt * TILE_N:(nt + 1) * TILE_N].astype(jnp.bfloat16)
            return jnp.dot(xj, wt, preferred_element_type=jnp.float32)

        def payload_rdma(t, nt):
            return pltpu.make_async_remote_copy(
                src_ref=send_ref, dst_ref=recv_ref.at[t, nt],
                send_sem=send_sem, recv_sem=recv_sems.at[t, nt],
                device_id=(right,), device_id_type=pl.DeviceIdType.MESH)

        def scale_rdma(t, nt):
            return pltpu.make_async_remote_copy(
                src_ref=sscale_ref, dst_ref=rscale_ref.at[t, nt],
                send_sem=sscale_sem, recv_sem=rscale_sems.at[t, nt],
                device_id=(right,), device_id_type=pl.DeviceIdType.MESH)

        def recv_payload_f32(t, nt):
            payload_rdma(t, nt).wait_recv()
            scale_rdma(t, nt).wait_recv()
            s = rscale_ref[t, nt, 0, 0]
            return recv_ref[t, nt].astype(jnp.float32) * s

        for t in range(n_hops):
            j = lax.rem(my + N_DEV - 1 - t, N_DEV)
            for nt in range(n_tiles):
                c = contrib_tile(j, nt)
                if t > 0:
                    c = c + recv_payload_f32(t - 1, nt)
                absmax = jnp.max(jnp.abs(c))
                inv = jnp.where(absmax > 0.0, 127.0 / absmax, 0.0)
                send_ref[...] = jnp.round(c * inv).astype(jnp.int8)
                sscale_ref[...] = jnp.full(
                    (8, 128), absmax * (1.0 / 127.0), jnp.float32)
                p = payload_rdma(t, nt)
                s = scale_rdma(t, nt)
                p.start()
                s.start()
                p.wait_send()
                s.wait_send()

        out_scale = sx_ref[0] * sw_ref[0]
        for nt in range(n_tiles):
            c = contrib_tile(my, nt) + recv_payload_f32(n_hops - 1, nt)
            stage_ref[...] = c * out_scale
            cp = pltpu.make_async_copy(
                stage_ref, out_ref.at[:, nt * TILE_N:(nt + 1) * TILE_N],
                copy_sem)
            cp.start()
            cp.wait()

    out_shape = jax.ShapeDtypeStruct((m_out, n), jnp.float32)
    return pl.pallas_call(
        body,
        out_shape=out_shape,
        in_specs=[
            pl.BlockSpec(memory_space=pltpu.VMEM),
            pl.BlockSpec(memory_space=pltpu.VMEM),
            pl.BlockSpec(memory_space=pltpu.SMEM),
            pl.BlockSpec(memory_space=pltpu.SMEM),
        ],
        out_specs=pl.BlockSpec(memory_space=pltpu.ANY),
        scratch_shapes=[
            pltpu.VMEM((m_out, TILE_N), jnp.int8),
            pltpu.VMEM((8, 128), jnp.float32),
            pltpu.VMEM((n_hops, n_tiles, m_out, TILE_N), jnp.int8),
            pltpu.VMEM((n_hops, n_tiles, 8, 128), jnp.float32),
            pltpu.VMEM((m_out, TILE_N), jnp.float32),
            pltpu.SemaphoreType.DMA,
            pltpu.SemaphoreType.DMA,
            pltpu.SemaphoreType.DMA((n_hops, n_tiles)),
            pltpu.SemaphoreType.DMA((n_hops, n_tiles)),
            pltpu.SemaphoreType.DMA,
        ],
        compiler_params=pltpu.CompilerParams(collective_id=0),
    )(x, w_mat, scale_x, scale_w)


# baseline (device time: 515686 ns/iter reference)
import jax
import jax.numpy as jnp
from jax import lax
from jax.experimental import pallas as pl
from jax.experimental.pallas import tpu as pltpu

N_DEV = 4
TILE_N = 1024


def kernel(x, w_mat, scale_x, scale_w):
    m, k = x.shape
    _kw, n = w_mat.shape
    m_out = m // N_DEV
    n_tiles = n // TILE_N
    n_hops = N_DEV - 1

    xb = x.astype(jnp.bfloat16)
    wb = w_mat.astype(jnp.bfloat16)

    def body(x_ref, w_ref, sx_ref, sw_ref, out_ref,
             xbuf, wbuf, send_ref, sscale_ref, recv_ref, rscale_ref,
             stage_ref, load_sems, send_sem, sscale_sem, recv_sems,
             rscale_sems, copy_sem):
        my = lax.axis_index("i")
        right = lax.rem(my + 1, N_DEV)
        left = lax.rem(my + N_DEV - 1, N_DEV)

        barrier = pltpu.get_barrier_semaphore()
        for nbr in (left, right):
            pl.semaphore_signal(barrier, inc=1, device_id=(nbr,),
                                device_id_type=pl.DeviceIdType.MESH)
        pl.semaphore_wait(barrier, 2)

        def load_x_chunk(j):
            cp = pltpu.make_async_copy(
                x_ref.at[pl.ds(j * m_out, m_out), :], xbuf, load_sems.at[0])
            cp.start()
            cp.wait()

        def load_w_tile(nt):
            cp = pltpu.make_async_copy(
                w_ref.at[:, pl.ds(nt * TILE_N, TILE_N)], wbuf,
                load_sems.at[1])
            cp.start()
            cp.wait()

        def payload_rdma(t, nt):
            return pltpu.make_async_remote_copy(
                src_ref=send_ref, dst_ref=recv_ref.at[t, nt],
                send_sem=send_sem, recv_sem=recv_sems.at[t, nt],
                device_id=(right,), device_id_type=pl.DeviceIdType.MESH)

        def scale_rdma(t, nt):
            return pltpu.make_async_remote_copy(
                src_ref=sscale_ref, dst_ref=rscale_ref.at[t, nt],
                send_sem=sscale_sem, recv_sem=rscale_sems.at[t, nt],
                device_id=(right,), device_id_type=pl.DeviceIdType.MESH)

        def recv_payload_f32(t, nt):
            payload_rdma(t, nt).wait_recv()
            scale_rdma(t, nt).wait_recv()
            s = rscale_ref[t, nt, 0, 0]
            return recv_ref[t, nt].astype(jnp.float32) * s

        for t in range(n_hops):
            j = lax.rem(my + N_DEV - 1 - t, N_DEV)
            load_x_chunk(j)
            for nt in range(n_tiles):
                load_w_tile(nt)
                c = jnp.dot(xbuf[...], wbuf[...],
                            preferred_element_type=jnp.float32)
                if t > 0:
                    c = c + recv_payload_f32(t - 1, nt)
                absmax = jnp.max(jnp.abs(c))
                inv = jnp.where(absmax > 0.0, 127.0 / absmax, 0.0)
                send_ref[...] = jnp.round(c * inv).astype(jnp.int8)
                sscale_ref[...] = jnp.full(
                    (8, 128), absmax * (1.0 / 127.0), jnp.float32)
                p = payload_rdma(t, nt)
                s = scale_rdma(t, nt)
                p.start()
                s.start()
                p.wait_send()
                s.wait_send()

        out_scale = sx_ref[0] * sw_ref[0]
        load_x_chunk(my)
        for nt in range(n_tiles):
            load_w_tile(nt)
            c = jnp.dot(xbuf[...], wbuf[...],
                        preferred_element_type=jnp.float32)
            c = c + recv_payload_f32(n_hops - 1, nt)
            stage_ref[...] = c * out_scale
            cp = pltpu.make_async_copy(
                stage_ref, out_ref.at[:, pl.ds(nt * TILE_N, TILE_N)],
                copy_sem)
            cp.start()
            cp.wait()

    out_shape = jax.ShapeDtypeStruct((m_out, n), jnp.float32)
    return pl.pallas_call(
        body,
        out_shape=out_shape,
        in_specs=[
            pl.BlockSpec(memory_space=pl.ANY),
            pl.BlockSpec(memory_space=pl.ANY),
            pl.BlockSpec(memory_space=pltpu.SMEM),
            pl.BlockSpec(memory_space=pltpu.SMEM),
        ],
        out_specs=pl.BlockSpec(memory_space=pl.ANY),
        scratch_shapes=[
            pltpu.VMEM((m_out, k), jnp.bfloat16),
            pltpu.VMEM((k, TILE_N), jnp.bfloat16),
            pltpu.VMEM((m_out, TILE_N), jnp.int8),
            pltpu.VMEM((8, 128), jnp.float32),
            pltpu.VMEM((n_hops, n_tiles, m_out, TILE_N), jnp.int8),
            pltpu.VMEM((n_hops, n_tiles, 8, 128), jnp.float32),
            pltpu.VMEM((m_out, TILE_N), jnp.float32),
            pltpu.SemaphoreType.DMA((2,)),
            pltpu.SemaphoreType.DMA,
            pltpu.SemaphoreType.DMA,
            pltpu.SemaphoreType.DMA((n_hops, n_tiles)),
            pltpu.SemaphoreType.DMA((n_hops, n_tiles)),
            pltpu.SemaphoreType.DMA,
        ],
        compiler_params=pltpu.CompilerParams(
            collective_id=0, vmem_limit_bytes=100 * 1024 * 1024),
    )(xb, wb, scale_x, scale_w)


# device time: 335704 ns/iter; 1.5361x vs baseline; 1.5361x over previous
import jax
import jax.numpy as jnp
from jax import lax
from jax.experimental import pallas as pl
from jax.experimental.pallas import tpu as pltpu

N_DEV = 4
TILE_N = 1024


def kernel(x, w_mat, scale_x, scale_w):
    m, k = x.shape
    _kw, n = w_mat.shape
    m_out = m // N_DEV
    n_tiles = n // TILE_N
    n_hops = N_DEV - 1
    n_items = N_DEV * n_tiles

    xb = x.astype(jnp.bfloat16)
    wb = w_mat.astype(jnp.bfloat16)

    def body(x_ref, w_ref, sx_ref, sw_ref, out_ref,
             xbuf, wbuf, send_ref, sscale_ref, recv_ref, rscale_ref,
             stage_ref, xsems, wsems, send_sems, sscale_sems, recv_sems,
             rscale_sems, copy_sems):
        my = lax.axis_index("i")
        right = lax.rem(my + 1, N_DEV)
        left = lax.rem(my + N_DEV - 1, N_DEV)

        barrier = pltpu.get_barrier_semaphore()
        for nbr in (left, right):
            pl.semaphore_signal(barrier, inc=1, device_id=(nbr,),
                                device_id_type=pl.DeviceIdType.MESH)
        pl.semaphore_wait(barrier, 2)

        def chunk_of_phase(t):
            return my if t == n_hops else lax.rem(my + N_DEV - 1 - t, N_DEV)

        def x_copy(t):
            j = chunk_of_phase(t)
            return pltpu.make_async_copy(
                x_ref.at[pl.ds(j * m_out, m_out), :], xbuf.at[t % 2],
                xsems.at[t % 2])

        def w_copy(p):
            nt = p % n_tiles
            return pltpu.make_async_copy(
                w_ref.at[:, pl.ds(nt * TILE_N, TILE_N)], wbuf.at[p % 2],
                wsems.at[p % 2])

        def payload_rdma(t, nt):
            s = (t * n_tiles + nt) % 2
            return pltpu.make_async_remote_copy(
                src_ref=send_ref.at[s], dst_ref=recv_ref.at[t, nt],
                send_sem=send_sems.at[s], recv_sem=recv_sems.at[t, nt],
                device_id=(right,), device_id_type=pl.DeviceIdType.MESH)

        def scale_rdma(t, nt):
            s = (t * n_tiles + nt) % 2
            return pltpu.make_async_remote_copy(
                src_ref=sscale_ref.at[s], dst_ref=rscale_ref.at[t, nt],
                send_sem=sscale_sems.at[s], recv_sem=rscale_sems.at[t, nt],
                device_id=(right,), device_id_type=pl.DeviceIdType.MESH)

        def recv_payload_f32(t, nt):
            payload_rdma(t, nt).wait_recv()
            scale_rdma(t, nt).wait_recv()
            s = rscale_ref[t, nt, 0, 0]
            return recv_ref[t, nt].astype(jnp.float32) * s

        out_scale = sx_ref[0] * sw_ref[0]
        stage_copy_pending = {}

        x_copy(0).start()
        w_copy(0).start()
        for t in range(N_DEV):
            x_copy(t).wait()
            if t + 1 < N_DEV:
                x_copy(t + 1).start()
            for nt in range(n_tiles):
                p = t * n_tiles + nt
                if p + 1 < n_items:
                    w_copy(p + 1).start()
                w_copy(p).wait()
                c = jnp.dot(xbuf[t % 2], wbuf[p % 2],
                            preferred_element_type=jnp.float32)
                if t > 0:
                    c = c + recv_payload_f32(t - 1, nt)
                if t < n_hops:
                    if p >= 2:
                        pq, sq = divmod(p - 2, n_tiles)
                        payload_rdma(pq, sq).wait_send()
                        scale_rdma(pq, sq).wait_send()
                    absmax = jnp.max(jnp.abs(c))
                    inv = jnp.where(absmax > 0.0, 127.0 / absmax, 0.0)
                    send_ref[p % 2] = jnp.round(c * inv).astype(jnp.int8)
                    sscale_ref[p % 2] = jnp.full(
                        (8, 128), absmax * (1.0 / 127.0), jnp.float32)
                    pr = payload_rdma(t, nt)
                    sr = scale_rdma(t, nt)
                    pr.start()
                    sr.start()
                else:
                    if nt >= 2:
                        stage_copy_pending[nt % 2].wait()
                    stage_ref[nt % 2] = c * out_scale
                    cp = pltpu.make_async_copy(
                        stage_ref.at[nt % 2],
                        out_ref.at[:, pl.ds(nt * TILE_N, TILE_N)],
                        copy_sems.at[nt % 2])
                    cp.start()
                    stage_copy_pending[nt % 2] = cp

        for p in (n_hops * n_tiles - 2, n_hops * n_tiles - 1):
            pq, sq = divmod(p, n_tiles)
            payload_rdma(pq, sq).wait_send()
            scale_rdma(pq, sq).wait_send()
        for s in range(2):
            stage_copy_pending[s].wait()

    out_shape = jax.ShapeDtypeStruct((m_out, n), jnp.float32)
    return pl.pallas_call(
        body,
        out_shape=out_shape,
        in_specs=[
            pl.BlockSpec(memory_space=pl.ANY),
            pl.BlockSpec(memory_space=pl.ANY),
            pl.BlockSpec(memory_space=pltpu.SMEM),
            pl.BlockSpec(memory_space=pltpu.SMEM),
        ],
        out_specs=pl.BlockSpec(memory_space=pl.ANY),
        scratch_shapes=[
            pltpu.VMEM((2, m_out, k), jnp.bfloat16),
            pltpu.VMEM((2, k, TILE_N), jnp.bfloat16),
            pltpu.VMEM((2, m_out, TILE_N), jnp.int8),
            pltpu.VMEM((2, 8, 128), jnp.float32),
            pltpu.VMEM((n_hops, n_tiles, m_out, TILE_N), jnp.int8),
            pltpu.VMEM((n_hops, n_tiles, 8, 128), jnp.float32),
            pltpu.VMEM((2, m_out, TILE_N), jnp.float32),
            pltpu.SemaphoreType.DMA((2,)),
            pltpu.SemaphoreType.DMA((2,)),
            pltpu.SemaphoreType.DMA((2,)),
            pltpu.SemaphoreType.DMA((2,)),
            pltpu.SemaphoreType.DMA((n_hops, n_tiles)),
            pltpu.SemaphoreType.DMA((n_hops, n_tiles)),
            pltpu.SemaphoreType.DMA((2,)),
        ],
        compiler_params=pltpu.CompilerParams(
            collective_id=0, vmem_limit_bytes=100 * 1024 * 1024),
    )(xb, wb, scale_x, scale_w)


# device time: 204374 ns/iter; 2.5232x vs baseline; 1.6426x over previous
import jax
import jax.numpy as jnp
from jax import lax
from jax.experimental import pallas as pl
from jax.experimental.pallas import tpu as pltpu

N_DEV = 4
TILE_N = 1024
N_HALF = 4


def kernel(x, w_mat, scale_x, scale_w):
    m, k = x.shape
    _kw, n = w_mat.shape
    m_out = m // N_DEV
    n_tiles = n // TILE_N
    n_hops = N_DEV - 1
    seq = [d * N_HALF + q for q in range(N_HALF) for d in (0, 1)]

    xb = x.astype(jnp.bfloat16)
    wb = w_mat.astype(jnp.bfloat16)

    def body(x_ref, w_ref, sx_ref, sw_ref, out_ref,
             xbuf, wbuf, send_ref, sscale_ref, recv_ref, rscale_ref,
             stage_ref, xsems, wsems, send_sems, sscale_sems, recv_sems,
             rscale_sems, copy_sems):
        my = lax.axis_index("i")
        right = lax.rem(my + 1, N_DEV)
        left = lax.rem(my + N_DEV - 1, N_DEV)

        barrier = pltpu.get_barrier_semaphore()
        for nbr in (left, right):
            pl.semaphore_signal(barrier, inc=1, device_id=(nbr,),
                                device_id_type=pl.DeviceIdType.MESH)
        pl.semaphore_wait(barrier, 2)

        def chunk_of(t, d):
            if t == n_hops:
                return my
            if d == 0:
                return lax.rem(my + N_DEV - 1 - t, N_DEV)
            return lax.rem(my + 1 + t, N_DEV)

        def x_copy(t, d):
            return pltpu.make_async_copy(
                x_ref.at[pl.ds(chunk_of(t, d) * m_out, m_out), :],
                xbuf.at[(t % 2) * 2 + d], xsems.at[(t % 2) * 2 + d])

        def w_copy(p):
            nt = seq[p % n_tiles]
            return pltpu.make_async_copy(
                w_ref.at[:, pl.ds(nt * TILE_N, TILE_N)], wbuf.at[p % 2],
                wsems.at[p % 2])

        def send_slot(t, d, q):
            return d * 2 + (t * N_HALF + q) % 2

        def payload_rdma(t, d, q):
            nt = d * N_HALF + q
            s = send_slot(t, d, q)
            return pltpu.make_async_remote_copy(
                src_ref=send_ref.at[s], dst_ref=recv_ref.at[t, nt],
                send_sem=send_sems.at[s], recv_sem=recv_sems.at[t, nt],
                device_id=(right if d == 0 else left,),
                device_id_type=pl.DeviceIdType.MESH)

        def scale_rdma(t, d, q):
            nt = d * N_HALF + q
            s = send_slot(t, d, q)
            return pltpu.make_async_remote_copy(
                src_ref=sscale_ref.at[s], dst_ref=rscale_ref.at[t, nt],
                send_sem=sscale_sems.at[s], recv_sem=rscale_sems.at[t, nt],
                device_id=(right if d == 0 else left,),
                device_id_type=pl.DeviceIdType.MESH)

        def recv_payload_f32(t, d, q):
            nt = d * N_HALF + q
            payload_rdma(t, d, q).wait_recv()
            scale_rdma(t, d, q).wait_recv()
            s = rscale_ref[t, nt, 0, 0]
            return recv_ref[t, nt].astype(jnp.float32) * s

        out_scale = sx_ref[0] * sw_ref[0]
        stage_copy_pending = {}

        x_copy(0, 0).start()
        x_copy(0, 1).start()
        w_copy(0).start()
        for t in range(N_DEV):
            x_copy(t, 0).wait()
            if t < n_hops:
                x_copy(t, 1).wait()
                if t + 1 < n_hops:
                    x_copy(t + 1, 0).start()
                    x_copy(t + 1, 1).start()
                else:
                    x_copy(t + 1, 0).start()
            for i in range(n_tiles):
                p = t * n_tiles + i
                nt = seq[i]
                d, q = divmod(nt, N_HALF)
                if p + 1 < N_DEV * n_tiles:
                    w_copy(p + 1).start()
                w_copy(p).wait()
                xslot = (t % 2) * 2 + (d if t < n_hops else 0)
                c = jnp.dot(xbuf[xslot], wbuf[p % 2],
                            preferred_element_type=jnp.float32)
                if t > 0:
                    c = c + recv_payload_f32(t - 1, d, q)
                if t < n_hops:
                    di = t * N_HALF + q
                    if di >= 2:
                        tq, qq = divmod(di - 2, N_HALF)
                        payload_rdma(tq, d, qq).wait_send()
                        scale_rdma(tq, d, qq).wait_send()
                    s = send_slot(t, d, q)
                    absmax = jnp.max(jnp.abs(c))
                    inv = jnp.where(absmax > 0.0, 127.0 / absmax, 0.0)
                    send_ref[s] = jnp.round(c * inv).astype(jnp.int8)
                    sscale_ref[s] = jnp.full(
                        (8, 128), absmax * (1.0 / 127.0), jnp.float32)
                    payload_rdma(t, d, q).start()
                    scale_rdma(t, d, q).start()
                else:
                    if i >= 2:
                        stage_copy_pending[i % 2].wait()
                    stage_ref[i % 2] = c * out_scale
                    cp = pltpu.make_async_copy(
                        stage_ref.at[i % 2],
                        out_ref.at[:, pl.ds(nt * TILE_N, TILE_N)],
                        copy_sems.at[i % 2])
                    cp.start()
                    stage_copy_pending[i % 2] = cp

        for d in (0, 1):
            for di in (n_hops * N_HALF - 2, n_hops * N_HALF - 1):
                tq, qq = divmod(di, N_HALF)
                payload_rdma(tq, d, qq).wait_send()
                scale_rdma(tq, d, qq).wait_send()
        for s in range(2):
            stage_copy_pending[s].wait()

    out_shape = jax.ShapeDtypeStruct((m_out, n), jnp.float32)
    return pl.pallas_call(
        body,
        out_shape=out_shape,
        in_specs=[
            pl.BlockSpec(memory_space=pl.ANY),
            pl.BlockSpec(memory_space=pl.ANY),
            pl.BlockSpec(memory_space=pltpu.SMEM),
            pl.BlockSpec(memory_space=pltpu.SMEM),
        ],
        out_specs=pl.BlockSpec(memory_space=pl.ANY),
        scratch_shapes=[
            pltpu.VMEM((4, m_out, k), jnp.bfloat16),
            pltpu.VMEM((2, k, TILE_N), jnp.bfloat16),
            pltpu.VMEM((4, m_out, TILE_N), jnp.int8),
            pltpu.VMEM((4, 8, 128), jnp.float32),
            pltpu.VMEM((n_hops, n_tiles, m_out, TILE_N), jnp.int8),
            pltpu.VMEM((n_hops, n_tiles, 8, 128), jnp.float32),
            pltpu.VMEM((2, m_out, TILE_N), jnp.float32),
            pltpu.SemaphoreType.DMA((4,)),
            pltpu.SemaphoreType.DMA((2,)),
            pltpu.SemaphoreType.DMA((4,)),
            pltpu.SemaphoreType.DMA((4,)),
            pltpu.SemaphoreType.DMA((n_hops, n_tiles)),
            pltpu.SemaphoreType.DMA((n_hops, n_tiles)),
            pltpu.SemaphoreType.DMA((2,)),
        ],
        compiler_params=pltpu.CompilerParams(
            collective_id=0, vmem_limit_bytes=100 * 1024 * 1024),
    )(xb, wb, scale_x, scale_w)


# device time: 196489 ns/iter; 2.6245x vs baseline; 1.0401x over previous
import jax
import jax.numpy as jnp
from jax import lax
from jax.experimental import pallas as pl
from jax.experimental.pallas import tpu as pltpu

N_DEV = 4
TILE_N = 1024
N_HALF = 4


def kernel(x, w_mat, scale_x, scale_w):
    m, k = x.shape
    _kw, n = w_mat.shape
    m_out = m // N_DEV
    n_tiles = n // TILE_N
    n_hops = N_DEV - 1
    seq = [d * N_HALF + q for q in range(N_HALF) for d in (0, 1)]

    xb = x.astype(jnp.float8_e4m3fn)
    wb = w_mat.astype(jnp.float8_e4m3fn)

    def body(x_ref, w_ref, sx_ref, sw_ref, out_ref,
             xbuf, wbuf, send_ref, sscale_ref, recv_ref, rscale_ref,
             stage_ref, xsems, wsems, send_sems, sscale_sems, recv_sems,
             rscale_sems, copy_sems):
        my = lax.axis_index("i")
        right = lax.rem(my + 1, N_DEV)
        left = lax.rem(my + N_DEV - 1, N_DEV)

        barrier = pltpu.get_barrier_semaphore()
        for nbr in (left, right):
            pl.semaphore_signal(barrier, inc=1, device_id=(nbr,),
                                device_id_type=pl.DeviceIdType.MESH)
        pl.semaphore_wait(barrier, 2)

        def chunk_of(t, d):
            if t == n_hops:
                return my
            if d == 0:
                return lax.rem(my + N_DEV - 1 - t, N_DEV)
            return lax.rem(my + 1 + t, N_DEV)

        def x_copy(t, d):
            return pltpu.make_async_copy(
                x_ref.at[pl.ds(chunk_of(t, d) * m_out, m_out), :],
                xbuf.at[(t % 2) * 2 + d], xsems.at[(t % 2) * 2 + d])

        def w_copy(p):
            nt = seq[p % n_tiles]
            return pltpu.make_async_copy(
                w_ref.at[:, pl.ds(nt * TILE_N, TILE_N)], wbuf.at[p % 2],
                wsems.at[p % 2])

        def send_slot(t, d, q):
            return d * 2 + (t * N_HALF + q) % 2

        def payload_rdma(t, d, q):
            nt = d * N_HALF + q
            s = send_slot(t, d, q)
            return pltpu.make_async_remote_copy(
                src_ref=send_ref.at[s], dst_ref=recv_ref.at[t, nt],
                send_sem=send_sems.at[s], recv_sem=recv_sems.at[t, nt],
                device_id=(right if d == 0 else left,),
                device_id_type=pl.DeviceIdType.MESH)

        def scale_rdma(t, d, q):
            nt = d * N_HALF + q
            s = send_slot(t, d, q)
            return pltpu.make_async_remote_copy(
                src_ref=sscale_ref.at[s], dst_ref=rscale_ref.at[t, nt],
                send_sem=sscale_sems.at[s], recv_sem=rscale_sems.at[t, nt],
                device_id=(right if d == 0 else left,),
                device_id_type=pl.DeviceIdType.MESH)

        def recv_payload_f32(t, d, q):
            nt = d * N_HALF + q
            payload_rdma(t, d, q).wait_recv()
            scale_rdma(t, d, q).wait_recv()
            s = rscale_ref[t, nt, 0, 0]
            return recv_ref[t, nt].astype(jnp.float32) * s

        out_scale = sx_ref[0] * sw_ref[0]
        stage_copy_pending = {}

        x_copy(0, 0).start()
        x_copy(0, 1).start()
        w_copy(0).start()
        for t in range(N_DEV):
            x_copy(t, 0).wait()
            if t < n_hops:
                x_copy(t, 1).wait()
                if t + 1 < n_hops:
                    x_copy(t + 1, 0).start()
                    x_copy(t + 1, 1).start()
                else:
                    x_copy(t + 1, 0).start()
            for i in range(n_tiles):
                p = t * n_tiles + i
                nt = seq[i]
                d, q = divmod(nt, N_HALF)
                if p + 1 < N_DEV * n_tiles:
                    w_copy(p + 1).start()
                w_copy(p).wait()
                xslot = (t % 2) * 2 + (d if t < n_hops else 0)
                c = jnp.dot(xbuf[xslot], wbuf[p % 2],
                            preferred_element_type=jnp.float32)
                if t > 0:
                    c = c + recv_payload_f32(t - 1, d, q)
                if t < n_hops:
                    di = t * N_HALF + q
                    if di >= 2:
                        tq, qq = divmod(di - 2, N_HALF)
                        payload_rdma(tq, d, qq).wait_send()
                        scale_rdma(tq, d, qq).wait_send()
                    s = send_slot(t, d, q)
                    absmax = jnp.max(jnp.abs(c))
                    inv = jnp.where(absmax > 0.0, 127.0 / absmax, 0.0)
                    send_ref[s] = jnp.round(c * inv).astype(jnp.int8)
                    sscale_ref[s] = jnp.full(
                        (8, 128), absmax * (1.0 / 127.0), jnp.float32)
                    payload_rdma(t, d, q).start()
                    scale_rdma(t, d, q).start()
                else:
                    if i >= 2:
                        stage_copy_pending[i % 2].wait()
                    stage_ref[i % 2] = c * out_scale
                    cp = pltpu.make_async_copy(
                        stage_ref.at[i % 2],
                        out_ref.at[:, pl.ds(nt * TILE_N, TILE_N)],
                        copy_sems.at[i % 2])
                    cp.start()
                    stage_copy_pending[i % 2] = cp

        for d in (0, 1):
            for di in (n_hops * N_HALF - 2, n_hops * N_HALF - 1):
                tq, qq = divmod(di, N_HALF)
                payload_rdma(tq, d, qq).wait_send()
                scale_rdma(tq, d, qq).wait_send()
        for s in range(2):
            stage_copy_pending[s].wait()

    out_shape = jax.ShapeDtypeStruct((m_out, n), jnp.float32)
    return pl.pallas_call(
        body,
        out_shape=out_shape,
        in_specs=[
            pl.BlockSpec(memory_space=pl.ANY),
            pl.BlockSpec(memory_space=pl.ANY),
            pl.BlockSpec(memory_space=pltpu.SMEM),
            pl.BlockSpec(memory_space=pltpu.SMEM),
        ],
        out_specs=pl.BlockSpec(memory_space=pl.ANY),
        scratch_shapes=[
            pltpu.VMEM((4, m_out, k), jnp.float8_e4m3fn),
            pltpu.VMEM((2, k, TILE_N), jnp.float8_e4m3fn),
            pltpu.VMEM((4, m_out, TILE_N), jnp.int8),
            pltpu.VMEM((4, 8, 128), jnp.float32),
            pltpu.VMEM((n_hops, n_tiles, m_out, TILE_N), jnp.int8),
            pltpu.VMEM((n_hops, n_tiles, 8, 128), jnp.float32),
            pltpu.VMEM((2, m_out, TILE_N), jnp.float32),
            pltpu.SemaphoreType.DMA((4,)),
            pltpu.SemaphoreType.DMA((2,)),
            pltpu.SemaphoreType.DMA((4,)),
            pltpu.SemaphoreType.DMA((4,)),
            pltpu.SemaphoreType.DMA((n_hops, n_tiles)),
            pltpu.SemaphoreType.DMA((n_hops, n_tiles)),
            pltpu.SemaphoreType.DMA((2,)),
        ],
        compiler_params=pltpu.CompilerParams(
            collective_id=0, vmem_limit_bytes=100 * 1024 * 1024),
    )(xb, wb, scale_x, scale_w)
